# baseline (speedup 1.0000x reference)
import functools
import math

import jax
import jax.numpy as jnp
from jax import lax
from jax.experimental import pallas as pl
from jax.experimental.pallas import tpu as pltpu
from jax.experimental.pallas import tpu_sc as plsc

EMB = 64
PAD = 128
SCALE = math.sqrt(EMB)
NC = 2
NS = 16
NW = NC * NS
LANES = 16
BBLK = 128
TPITCH = 133


@functools.lru_cache(maxsize=None)
def _make(batch, seq, vocab):
    assert batch == NW * BBLK

    @functools.partial(
        pl.kernel,
        out_type=jax.ShapeDtypeStruct((seq, EMB, batch), jnp.float32),
        mesh=plsc.VectorSubcoreMesh(
            core_axis_name="c", subcore_axis_name="s",
            num_cores=NC, num_subcores=NS,
        ),
        scratch_types=[
            pltpu.VMEM((seq, BBLK), jnp.int32),
            pltpu.VMEM((BBLK, PAD), jnp.float32),
            pltpu.VMEM((EMB, TPITCH), jnp.float32),
            pltpu.SemaphoreType.DMA,
        ],
        compiler_params=pltpu.CompilerParams(
            use_tc_tiling_on_sc=True, needs_layout_passes=False,
        ),
    )
    def emb_kernel(tokens_hbm, table_hbm, out_hbm, idx_v, rows_v, tr_v, gsem):
        wid = lax.axis_index("s") * NC + lax.axis_index("c")
        b0 = wid * BBLK
        pltpu.sync_copy(tokens_hbm.at[:, pl.ds(b0, BBLK)], idx_v)

        ridx = [e0 + lax.iota(jnp.int32, 16) for e0 in range(0, EMB, LANES)]

        @pl.loop(0, seq)
        def _s(s):
            pltpu.async_copy(table_hbm.at[idx_v.at[s]], rows_v, gsem).wait()

            @pl.loop(0, BBLK)
            def _b(b):
                cidx = jnp.full((16,), b, jnp.int32)
                for k in range(EMB // LANES):
                    v = rows_v[b, pl.ds(k * LANES, LANES)] * SCALE
                    plsc.store_scatter(tr_v, [ridx[k], cidx], v)

            pltpu.sync_copy(
                tr_v.at[:, pl.ds(0, BBLK)],
                out_hbm.at[s, :, pl.ds(b0, BBLK)],
            )

    return emb_kernel


def kernel(tokens, embedding):
    b, s = tokens.shape
    tokens_t = jnp.swapaxes(tokens, 0, 1)
    table_p = jnp.pad(embedding, ((0, 0), (0, PAD - EMB)))
    out_phys = _make(b, s, embedding.shape[0])(tokens_t, table_p)
    return jnp.transpose(out_phys, (2, 0, 1))


# speedup vs baseline: 1.2197x; 1.2197x over previous
import functools
import math

import jax
import jax.numpy as jnp
from jax import lax
from jax.experimental import pallas as pl
from jax.experimental.pallas import tpu as pltpu
from jax.experimental.pallas import tpu_sc as plsc

EMB = 64
PAD = 128
SCALE = math.sqrt(EMB)
NC = 2
NS = 16
NW = NC * NS
LANES = 16
BBLK = 128
TPITCH = 133
NRING = 4


@functools.lru_cache(maxsize=None)
def _make(batch, seq, vocab):
    assert batch == NW * BBLK
    assert seq % NRING == 0

    @functools.partial(
        pl.kernel,
        out_type=jax.ShapeDtypeStruct((seq, EMB, batch), jnp.float32),
        mesh=plsc.VectorSubcoreMesh(
            core_axis_name="c", subcore_axis_name="s",
            num_cores=NC, num_subcores=NS,
        ),
        scratch_types=[
            pltpu.VMEM((seq, BBLK), jnp.int32),
            pltpu.VMEM((NRING, BBLK, PAD), jnp.float32),
            pltpu.VMEM((2, EMB, TPITCH), jnp.float32),
            pltpu.SemaphoreType.DMA,
            pltpu.SemaphoreType.DMA,
            pltpu.SemaphoreType.DMA,
            pltpu.SemaphoreType.DMA,
            pltpu.SemaphoreType.DMA,
            pltpu.SemaphoreType.DMA,
        ],
        compiler_params=pltpu.CompilerParams(
            use_tc_tiling_on_sc=True, needs_layout_passes=False,
        ),
    )
    def emb_kernel(tokens_hbm, table_hbm, out_hbm, idx_v, rows_v, tr_v,
                   gs0, gs1, gs2, gs3, ws0, ws1):
        gsems = [gs0, gs1, gs2, gs3]
        wsems = [ws0, ws1]
        wid = lax.axis_index("s") * NC + lax.axis_index("c")
        b0 = wid * BBLK
        pltpu.sync_copy(tokens_hbm.at[:, pl.ds(b0, BBLK)], idx_v)

        ridx = [e0 + lax.iota(jnp.int32, 16) for e0 in range(0, EMB, LANES)]

        def start_gather(s, p):
            pltpu.async_copy(table_hbm.at[idx_v.at[s]], rows_v.at[p], gsems[p])

        def wait_gather(p):
            pltpu.make_async_copy(
                table_hbm.at[idx_v.at[0]], rows_v.at[p], gsems[p]
            ).wait()

        def start_write(s, q):
            pltpu.async_copy(
                tr_v.at[q, :, pl.ds(0, BBLK)],
                out_hbm.at[s, :, pl.ds(b0, BBLK)],
                wsems[q],
            )

        def wait_write(q):
            pltpu.make_async_copy(
                tr_v.at[q, :, pl.ds(0, BBLK)],
                out_hbm.at[0, :, pl.ds(b0, BBLK)],
                wsems[q],
            ).wait()

        for p in range(NRING - 1):
            start_gather(p, p)

        @pl.loop(0, seq // NRING)
        def _g(g):
            s4 = g * NRING
            for p in range(NRING):
                s = s4 + p
                q = p % 2
                wait_gather(p)

                @pl.when(s >= 2)
                def _():
                    wait_write(q)

                @pl.loop(0, BBLK, step=8)
                def _b(b):
                    for u in range(8):
                        bb = b + u
                        cidx = jnp.full((16,), bb, jnp.int32)
                        for k in range(EMB // LANES):
                            v = rows_v[p, bb, pl.ds(k * LANES, LANES)] * SCALE
                            plsc.store_scatter(tr_v.at[q], [ridx[k], cidx], v)

                start_write(s, q)

                @pl.when(s + NRING - 1 < seq)
                def _():
                    start_gather(s + NRING - 1, (p + NRING - 1) % NRING)

        wait_write(0)
        wait_write(1)

    return emb_kernel


def kernel(tokens, embedding):
    b, s = tokens.shape
    tokens_t = jnp.swapaxes(tokens, 0, 1)
    table_p = jnp.pad(embedding, ((0, 0), (0, PAD - EMB)))
    out_phys = _make(b, s, embedding.shape[0])(tokens_t, table_p)
    return jnp.transpose(out_phys, (2, 0, 1))


# speedup vs baseline: 1.2281x; 1.0069x over previous
import functools
import math

import jax
import jax.numpy as jnp
from jax import lax
from jax.experimental import pallas as pl
from jax.experimental.pallas import tpu as pltpu
from jax.experimental.pallas import tpu_sc as plsc

EMB = 64
PAD = 128
SCALE = math.sqrt(EMB)
NC = 2
NS = 16
NW = NC * NS
LANES = 16
BBLK = 128
TPITCH = 133
NRING = 4


@functools.lru_cache(maxsize=None)
def _make(batch, seq, vocab):
    assert batch == NW * BBLK
    assert seq % NRING == 0

    @functools.partial(
        pl.kernel,
        out_type=jax.ShapeDtypeStruct((seq, EMB, batch), jnp.float32),
        mesh=plsc.VectorSubcoreMesh(
            core_axis_name="c", subcore_axis_name="s",
            num_cores=NC, num_subcores=NS,
        ),
        scratch_types=[
            pltpu.VMEM((seq, BBLK), jnp.int32),
            pltpu.VMEM((NRING, BBLK, PAD), jnp.float32),
            pltpu.VMEM((2, EMB, TPITCH), jnp.float32),
            pltpu.SemaphoreType.DMA,
            pltpu.SemaphoreType.DMA,
            pltpu.SemaphoreType.DMA,
            pltpu.SemaphoreType.DMA,
            pltpu.SemaphoreType.DMA,
            pltpu.SemaphoreType.DMA,
        ],
        compiler_params=pltpu.CompilerParams(
            use_tc_tiling_on_sc=True, needs_layout_passes=False,
        ),
    )
    def emb_kernel(tokens_hbm, table_hbm, out_hbm, idx_v, rows_v, tr_v,
                   gs0, gs1, gs2, gs3, ws0, ws1):
        gsems = [gs0, gs1, gs2, gs3]
        wsems = [ws0, ws1]
        wid = lax.axis_index("s") * NC + lax.axis_index("c")
        b0 = wid * BBLK
        pltpu.sync_copy(tokens_hbm.at[:, pl.ds(b0, BBLK)], idx_v)

        ridx = [e0 + lax.iota(jnp.int32, 16) for e0 in range(0, EMB, LANES)]

        def start_gather(s, p):
            pltpu.async_copy(table_hbm.at[idx_v.at[s]], rows_v.at[p], gsems[p])

        def wait_gather(p):
            pltpu.make_async_copy(
                table_hbm.at[idx_v.at[0]], rows_v.at[p], gsems[p]
            ).wait()

        def start_write(s, q):
            pltpu.async_copy(
                tr_v.at[q, :, pl.ds(0, BBLK)],
                out_hbm.at[s, :, pl.ds(b0, BBLK)],
                wsems[q],
            )

        def wait_write(q):
            pltpu.make_async_copy(
                tr_v.at[q, :, pl.ds(0, BBLK)],
                out_hbm.at[0, :, pl.ds(b0, BBLK)],
                wsems[q],
            ).wait()

        for p in range(NRING - 1):
            start_gather(p, p)

        @pl.loop(0, seq // NRING)
        def _g(g):
            s4 = g * NRING
            for p in range(NRING):
                s = s4 + p
                q = p % 2
                wait_gather(p)

                @pl.when(s >= 2)
                def _():
                    wait_write(q)

                @pl.loop(0, BBLK, step=16)
                def _b(b):
                    for u in range(16):
                        bb = b + u
                        cidx = jnp.full((16,), bb, jnp.int32)
                        for k in range(EMB // LANES):
                            v = rows_v[p, bb, pl.ds(k * LANES, LANES)] * SCALE
                            plsc.store_scatter(tr_v.at[q], [ridx[k], cidx], v)

                start_write(s, q)

                @pl.when(s + NRING - 1 < seq)
                def _():
                    start_gather(s + NRING - 1, (p + NRING - 1) % NRING)

        wait_write(0)
        wait_write(1)

    return emb_kernel


def kernel(tokens, embedding):
    b, s = tokens.shape
    tokens_t = jnp.swapaxes(tokens, 0, 1)
    table_p = jnp.pad(embedding, ((0, 0), (0, PAD - EMB)))
    out_phys = _make(b, s, embedding.shape[0])(tokens_t, table_p)
    return jnp.transpose(out_phys, (2, 0, 1))


# speedup vs baseline: 1.5506x; 1.2625x over previous
import functools
import math

import jax
import jax.numpy as jnp
from jax import lax
from jax.experimental import pallas as pl
from jax.experimental.pallas import tpu as pltpu
from jax.experimental.pallas import tpu_sc as plsc

EMB = 64
SCALE = math.sqrt(EMB)
NC = 2
NS = 16
NW = NC * NS
CHUNK = 800
LANES = 16


@functools.lru_cache(maxsize=None)
def _make(n_tokens, vocab, interpret=False):
    assert n_tokens % NW == 0
    per_w = n_tokens // NW
    assert per_w % CHUNK == 0
    n_chunks = per_w // CHUNK

    @functools.partial(
        pl.kernel,
        out_type=jax.ShapeDtypeStruct((n_tokens, EMB), jnp.float32),
        mesh=plsc.VectorSubcoreMesh(
            core_axis_name="c", subcore_axis_name="s",
            num_cores=NC, num_subcores=NS,
        ),
        scratch_types=[
            pltpu.VMEM((per_w,), jnp.int32),
            pltpu.VMEM((CHUNK, EMB), jnp.float32),
            pltpu.SemaphoreType.DMA,
        ],
        compiler_params=pltpu.CompilerParams(use_tc_tiling_on_sc=False),
        interpret=interpret,
    )
    def emb_kernel(tokens_hbm, table_hbm, out_hbm, idx_v, rows_v, gsem):
        wid = lax.axis_index("s") * NC + lax.axis_index("c")
        base = wid * per_w
        pltpu.sync_copy(tokens_hbm.at[pl.ds(base, per_w)], idx_v)

        @pl.loop(0, n_chunks)
        def _chunk(c):
            off = c * CHUNK
            pltpu.async_copy(
                table_hbm.at[idx_v.at[pl.ds(off, CHUNK)]], rows_v, gsem
            ).wait()

            @pl.loop(0, CHUNK, step=4)
            def _scale(i):
                for t in range(4):
                    for j in range(EMB // LANES):
                        sl = (i + t, pl.ds(j * LANES, LANES))
                        rows_v[sl] = rows_v[sl] * SCALE

            pltpu.sync_copy(rows_v, out_hbm.at[pl.ds(base + off, CHUNK)])

    return emb_kernel


def kernel(tokens, embedding):
    b, s = tokens.shape
    flat = tokens.reshape(-1).astype(jnp.int32)
    out = _make(b * s, embedding.shape[0])(flat, embedding)
    return out.reshape(b, s, EMB)


# speedup vs baseline: 1.6424x; 1.0593x over previous
import functools
import math

import jax
import jax.numpy as jnp
from jax import lax
from jax.experimental import pallas as pl
from jax.experimental.pallas import tpu as pltpu
from jax.experimental.pallas import tpu_sc as plsc

EMB = 64
SCALE = math.sqrt(EMB)
NC = 2
NS = 16
NW = NC * NS
CHUNK = 400
LANES = 16
NRING = 4


@functools.lru_cache(maxsize=None)
def _make(n_tokens, vocab):
    assert n_tokens % NW == 0
    per_w = n_tokens // NW
    assert per_w % (CHUNK * NRING) == 0
    n_chunks = per_w // CHUNK

    @functools.partial(
        pl.kernel,
        out_type=jax.ShapeDtypeStruct((n_tokens, EMB), jnp.float32),
        mesh=plsc.VectorSubcoreMesh(
            core_axis_name="c", subcore_axis_name="s",
            num_cores=NC, num_subcores=NS,
        ),
        scratch_types=[
            pltpu.VMEM((per_w,), jnp.int32),
            pltpu.VMEM((NRING, CHUNK, EMB), jnp.float32),
            pltpu.SemaphoreType.DMA,
            pltpu.SemaphoreType.DMA,
            pltpu.SemaphoreType.DMA,
            pltpu.SemaphoreType.DMA,
            pltpu.SemaphoreType.DMA,
            pltpu.SemaphoreType.DMA,
            pltpu.SemaphoreType.DMA,
            pltpu.SemaphoreType.DMA,
        ],
        compiler_params=pltpu.CompilerParams(use_tc_tiling_on_sc=False),
    )
    def emb_kernel(tokens_hbm, table_hbm, out_hbm, idx_v, rows_v,
                   gs0, gs1, gs2, gs3, ws0, ws1, ws2, ws3):
        gsems = [gs0, gs1, gs2, gs3]
        wsems = [ws0, ws1, ws2, ws3]
        wid = lax.axis_index("s") * NC + lax.axis_index("c")
        base = wid * per_w
        pltpu.sync_copy(tokens_hbm.at[pl.ds(base, per_w)], idx_v)

        def start_gather(c, p):
            pltpu.async_copy(
                table_hbm.at[idx_v.at[pl.ds(c * CHUNK, CHUNK)]],
                rows_v.at[p], gsems[p],
            )

        def wait_gather(p):
            pltpu.make_async_copy(
                table_hbm.at[idx_v.at[pl.ds(0, CHUNK)]], rows_v.at[p], gsems[p]
            ).wait()

        def start_write(c, p):
            pltpu.async_copy(
                rows_v.at[p], out_hbm.at[pl.ds(base + c * CHUNK, CHUNK)],
                wsems[p],
            )

        def wait_write(p):
            pltpu.make_async_copy(
                rows_v.at[p], out_hbm.at[pl.ds(base, CHUNK)], wsems[p]
            ).wait()

        for p in range(NRING - 1):
            start_gather(p, p)

        @pl.loop(0, n_chunks // NRING)
        def _g(g):
            c4 = g * NRING
            for p in range(NRING):
                c = c4 + p
                wait_gather(p)

                @pl.loop(0, CHUNK, step=4)
                def _scale(i):
                    for t in range(4):
                        for j in range(EMB // LANES):
                            sl = (p, i + t, pl.ds(j * LANES, LANES))
                            rows_v[sl] = rows_v[sl] * SCALE

                start_write(c, p)

                @pl.when(jnp.logical_and(c >= 1, c + NRING - 1 < n_chunks))
                def _():
                    wait_write((p + NRING - 1) % NRING)

                @pl.when(c + NRING - 1 < n_chunks)
                def _():
                    start_gather(c + NRING - 1, (p + NRING - 1) % NRING)

        for p in range(NRING):
            wait_write(p)

    return emb_kernel


def kernel(tokens, embedding):
    b, s = tokens.shape
    flat = tokens.reshape(-1).astype(jnp.int32)
    out = _make(b * s, embedding.shape[0])(flat, embedding)
    return out.reshape(b, s, EMB)
